# baseline (speedup 1.0000x reference)
import functools

import jax
import jax.numpy as jnp
from jax import lax
from jax.experimental import pallas as pl
from jax.experimental.pallas import tpu as pltpu
from jax.experimental.pallas import tpu_sc as plsc

_NC, _NS, _L = 2, 16, 16


def _tc_copy_body(kb_hbm, vb_hbm, out_hbm, *, m, h, d, bm):
    nblk = m // bm

    def inner(src_ref, dst_ref):
        dst_ref[0] = src_ref[...]

    for src, half in ((kb_hbm, 0), (vb_hbm, 1)):
        pltpu.emit_pipeline(
            inner,
            grid=(nblk,),
            in_specs=[pl.BlockSpec((bm, h, d), lambda i: (i, 0, 0))],
            out_specs=[
                pl.BlockSpec((1, bm, h, d),
                             lambda i, _s=half: (_s, i, 0, 0))
            ],
        )(src, out_hbm)


def _sc_scatter_body(k_hbm, v_hbm, idx_hbm, out_hbm, idx_v, sem, *, nb, rows):
    wid = lax.axis_index("s") * _NC + lax.axis_index("c")
    base = wid * rows
    pltpu.sync_copy(idx_hbm, idx_v)
    nchunk = nb // _L
    iota = lax.iota(jnp.int32, _L)
    chunks = [idx_v[pl.ds(c * _L, _L)] for c in range(nchunk)]

    hit_cnt = []
    for c in range(nchunk):
        hm = (chunks[c] >= base) & (chunks[c] < base + rows)
        hit_cnt.append(plsc.all_reduce_population_count(hm)[0])

    def _ok(c, j, t):
        tv = jnp.full((_L,), t, dtype=jnp.int32)
        later = jnp.int32(0)
        for c2 in range(c, nchunk):
            eq = chunks[c2] == tv
            if c2 == c:
                eq = eq & (iota > j)
            later = later + plsc.all_reduce_population_count(eq)[0]
        return (t >= base) & (t < base + rows) & (later == 0)

    def _copies(b, t):
        kcp = pltpu.make_async_copy(k_hbm.at[pl.ds(b, 1)],
                                    out_hbm.at[0, pl.ds(t, 1)], sem)
        vcp = pltpu.make_async_copy(v_hbm.at[pl.ds(b, 1)],
                                    out_hbm.at[1, pl.ds(t, 1)], sem)
        return kcp, vcp

    for phase in ("start", "wait"):
        for c in range(nchunk):
            @pl.when(hit_cnt[c] > 0)
            def _(c=c, phase=phase):
                for j in range(_L):
                    t = chunks[c][j]

                    @pl.when(_ok(c, j, t))
                    def _(j=j, t=t):
                        kcp, vcp = _copies(c * _L + j, t)
                        if phase == "start":
                            kcp.start()
                            vcp.start()
                        else:
                            kcp.wait()
                            vcp.wait()


def kernel(k, v, mem_index, key_buffer, value_buffer):
    m, h, d = key_buffer.shape
    nb = k.shape[0]

    out_ref = jax.empty_ref(
        jax.ShapeDtypeStruct((2, m, h, d), key_buffer.dtype))

    tc_mesh = pltpu.create_tensorcore_mesh("x", num_cores=1)
    copy_body = functools.partial(_tc_copy_body, m=m, h=h, d=d, bm=1024)
    pl.kernel(
        copy_body,
        out_type=(),
        mesh=tc_mesh,
    )(key_buffer, value_buffer, out_ref)

    rows = m // (_NC * _NS)
    scatter_body = functools.partial(_sc_scatter_body, nb=nb, rows=rows)
    sc_mesh = plsc.VectorSubcoreMesh(core_axis_name="c", subcore_axis_name="s")
    pl.kernel(
        scatter_body,
        out_type=(),
        mesh=sc_mesh,
        compiler_params=pltpu.CompilerParams(needs_layout_passes=False),
        scratch_types=[pltpu.VMEM((nb,), jnp.int32),
                       pltpu.SemaphoreType.DMA],
    )(k.reshape(nb, h, d), v.reshape(nb, h, d),
      mem_index.astype(jnp.int32), out_ref)

    return jax.freeze(out_ref)


# speedup vs baseline: 1.0191x; 1.0191x over previous
import functools

import jax
import jax.numpy as jnp
from jax import lax
from jax.experimental import pallas as pl
from jax.experimental.pallas import tpu as pltpu
from jax.experimental.pallas import tpu_sc as plsc

_NC, _NS, _L = 2, 16, 16


def _tc_copy_body(kb_ref, vb_ref, out_ref):
    out_ref[0] = kb_ref[...]
    out_ref[1] = vb_ref[...]


def _sc_scatter_body(k_hbm, v_hbm, idx_hbm, out_hbm, idx_v, sem, *, nb, rows):
    wid = lax.axis_index("s") * _NC + lax.axis_index("c")
    base = wid * rows
    pltpu.sync_copy(idx_hbm, idx_v)
    nchunk = nb // _L
    iota = lax.iota(jnp.int32, _L)
    chunks = [idx_v[pl.ds(c * _L, _L)] for c in range(nchunk)]

    oks, ts = [], []
    for c in range(nchunk):
        for j in range(_L):
            t = chunks[c][j]
            tv = jnp.full((_L,), t, dtype=jnp.int32)
            later = jnp.int32(0)
            for c2 in range(c, nchunk):
                eq = chunks[c2] == tv
                if c2 == c:
                    eq = eq & (iota > j)
                later = later + plsc.all_reduce_population_count(eq)[0]
            oks.append((t >= base) & (t < base + rows) & (later == 0))
            ts.append(t)

    def _copies(b):
        t = ts[b]
        kcp = pltpu.make_async_copy(k_hbm.at[pl.ds(b, 1)],
                                    out_hbm.at[0, pl.ds(t, 1)], sem)
        vcp = pltpu.make_async_copy(v_hbm.at[pl.ds(b, 1)],
                                    out_hbm.at[1, pl.ds(t, 1)], sem)
        return kcp, vcp

    for b in range(nb):
        @pl.when(oks[b])
        def _():
            kcp, vcp = _copies(b)
            kcp.start()
            vcp.start()

    for b in range(nb):
        @pl.when(oks[b])
        def _():
            kcp, vcp = _copies(b)
            kcp.wait()
            vcp.wait()


def kernel(k, v, mem_index, key_buffer, value_buffer):
    m, h, d = key_buffer.shape
    nb = k.shape[0]

    bm = min(1024, m)
    out = pl.pallas_call(
        _tc_copy_body,
        grid=(m // bm,),
        in_specs=[
            pl.BlockSpec((bm, h, d), lambda i: (i, 0, 0)),
            pl.BlockSpec((bm, h, d), lambda i: (i, 0, 0)),
        ],
        out_specs=pl.BlockSpec((2, bm, h, d), lambda i: (0, i, 0, 0)),
        out_shape=jax.ShapeDtypeStruct((2, m, h, d), key_buffer.dtype),
    )(key_buffer, value_buffer)
    out_ref = jax.new_ref(out)

    rows = m // (_NC * _NS)
    scatter_body = functools.partial(_sc_scatter_body, nb=nb, rows=rows)
    sc_mesh = plsc.VectorSubcoreMesh(core_axis_name="c", subcore_axis_name="s")
    pl.kernel(
        scatter_body,
        out_type=(),
        mesh=sc_mesh,
        compiler_params=pltpu.CompilerParams(needs_layout_passes=False),
        scratch_types=[pltpu.VMEM((nb,), jnp.int32),
                       pltpu.SemaphoreType.DMA],
    )(k.reshape(nb, h, d), v.reshape(nb, h, d),
      mem_index.astype(jnp.int32), out_ref)

    return jax.freeze(out_ref)
